# Initial kernel scaffold; baseline (speedup 1.0000x reference)
import jax
import jax.numpy as jnp
from jax.experimental import pallas as pl


def kernel(n_feat, e_feat, edge_index, W_msg_w, W_msg_b, W_w, W_b, a):
    raise NotImplementedError("write your pallas kernel here")


# speedup vs baseline: 7.9758x; 7.9758x over previous
import functools

import jax
import jax.numpy as jnp
from jax import lax
from jax.experimental import pallas as pl
from jax.experimental.pallas import tpu as pltpu
from jax.experimental.pallas import tpu_sc as plsc

N = 10000
E = 320000
NF = 128
EF = 16
OF = 32
DIN = 2 * NF + EF

NC = 2
NS = 16
NW = NC * NS
EPT = E // NW
CH = 80
NCHUNK = EPT // CH
DZ = 640
DTAIL = N - (NS - 1) * DZ
DPAD = NS * DZ

BE = 10000

_f32 = jnp.float32


def _node_kernel(n_ref, ww_ref, wm_ref, wmb_ref, a_ref,
                 asrc_ref, adst_ref, msrc_ref, mdstb_ref):
    nf = n_ref[...]
    ww = ww_ref[...]
    wm = wm_ref[...]
    a2 = a_ref[...]
    dn = (((1,), (1,)), ((), ()))
    v_src = lax.dot_general(ww[:, :NF], a2, (((0,), (0,)), ((), ())),
                            preferred_element_type=_f32)
    v_dst = lax.dot_general(ww[:, NF:2 * NF], a2, (((0,), (0,)), ((), ())),
                            preferred_element_type=_f32)
    asrc_ref[...] = lax.dot_general(nf, v_src, (((1,), (0,)), ((), ())),
                                    preferred_element_type=_f32)
    adst_ref[...] = lax.dot_general(nf, v_dst, (((1,), (0,)), ((), ())),
                                    preferred_element_type=_f32)
    msrc_ref[...] = lax.dot_general(nf, wm[:, :NF], dn,
                                    preferred_element_type=_f32)
    mdstb_ref[...] = lax.dot_general(nf, wm[:, NF:2 * NF], dn,
                                     preferred_element_type=_f32) + wmb_ref[...]


def _edge_kernel(e_ref, wm_ref, ww_ref, a_ref, wb_ref, me_ref, ae_ref):
    ef = e_ref[...]
    dn = (((1,), (1,)), ((), ()))
    me_ref[...] = lax.dot_general(ef, wm_ref[...][:, 2 * NF:], dn,
                                  preferred_element_type=_f32)
    v_e = lax.dot_general(ww_ref[...][:, 2 * NF:], a_ref[...],
                          (((0,), (0,)), ((), ())),
                          preferred_element_type=_f32)
    c = lax.dot_general(wb_ref[...], a_ref[...], (((1,), (0,)), ((), ())),
                        preferred_element_type=_f32)
    ae_ref[...] = lax.dot_general(ef, v_e, (((1,), (0,)), ((), ())),
                                  preferred_element_type=_f32) + c[0, 0]


def _combine_kernel(hp_ref, dp_ref, mdstb_ref, out_ref):
    hs = hp_ref[0] + hp_ref[1]
    dn = dp_ref[0] + dp_ref[1]
    valid = dn > 0.0
    safe = jnp.where(valid, dn, 1.0)
    h = jnp.where(valid, hs / safe + mdstb_ref[...], 0.0)
    out_ref[...] = jnp.maximum(h, 0.0)


_sc_mesh = plsc.VectorSubcoreMesh(core_axis_name="c", subcore_axis_name="s")


@functools.partial(
    pl.kernel,
    out_type=(jax.ShapeDtypeStruct((NC, N, OF), _f32),
              jax.ShapeDtypeStruct((NC, N), _f32)),
    mesh=_sc_mesh,
    scratch_types=[
        pltpu.VMEM_SHARED((N, OF), _f32),
        pltpu.VMEM_SHARED((DPAD,), _f32),
        pltpu.VMEM((N,), _f32),
        pltpu.VMEM((N,), _f32),
        pltpu.VMEM((CH,), jnp.int32),
        pltpu.VMEM((CH,), jnp.int32),
        pltpu.VMEM((CH,), _f32),
        pltpu.VMEM((CH,), _f32),
        pltpu.VMEM((CH, OF), _f32),
        pltpu.VMEM((CH, OF), _f32),
        pltpu.VMEM((DZ, OF), _f32),
        pltpu.VMEM((DZ,), _f32),
        pltpu.SemaphoreType.DMA,
    ],
    compiler_params=pltpu.CompilerParams(needs_layout_passes=False,
                                         use_tc_tiling_on_sc=False),
)
def _sc_gat(src_hbm, dst_hbm, asrc_hbm, adst_hbm, ae_hbm, msrc_hbm, me_hbm,
            h_out, d_out,
            h_sh, d_sh, asrc_v, adst_v, src_v, dst_v, ae_v, ex_v,
            rows_v, me_v, zh_v, zd_v, sem):
    core = lax.axis_index("c")
    sub = lax.axis_index("s")
    wid = core * NS + sub

    zvec = jnp.zeros((16,), _f32)

    def _zh(i, carry):
        zh_v[i, pl.ds(0, 16)] = zvec
        zh_v[i, pl.ds(16, 16)] = zvec
        return carry

    lax.fori_loop(0, DZ, _zh, 0)

    def _zd(i, carry):
        zd_v[pl.ds(i * 16, 16)] = zvec
        return carry

    lax.fori_loop(0, DZ // 16, _zd, 0)

    @pl.when(sub < NS - 1)
    def _():
        pltpu.sync_copy(zh_v, h_sh.at[pl.ds(sub * DZ, DZ), :])

    @pl.when(sub == NS - 1)
    def _():
        pltpu.sync_copy(zh_v.at[pl.ds(0, DTAIL), :],
                        h_sh.at[pl.ds((NS - 1) * DZ, DTAIL), :])

    pltpu.sync_copy(zd_v, d_sh.at[pl.ds(sub * DZ, DZ)])

    pltpu.sync_copy(asrc_hbm, asrc_v)
    pltpu.sync_copy(adst_hbm, adst_v)
    plsc.subcore_barrier()

    tile_base = wid * EPT

    def _chunk(chi, carry):
        base = tile_base + chi * CH
        pltpu.sync_copy(src_hbm.at[pl.ds(base, CH)], src_v)
        pltpu.sync_copy(dst_hbm.at[pl.ds(base, CH)], dst_v)
        pltpu.sync_copy(ae_hbm.at[pl.ds(base, CH)], ae_v)
        pltpu.sync_copy(me_hbm.at[pl.ds(base, CH), :], me_v)
        gat = pltpu.async_copy(msrc_hbm.at[src_v], rows_v, sem)
        for g in range(CH // 16):
            sl = pl.ds(g * 16, 16)
            a_s = plsc.load_gather(asrc_v, [src_v[sl]])
            a_d = plsc.load_gather(adst_v, [dst_v[sl]])
            z = a_s + a_d + ae_v[sl]
            lg = jnp.maximum(z, 0.01 * z)
            ex_v[sl] = jnp.exp(lg)
        pltpu.sync_copy(ex_v, d_sh.at[dst_v], add=True)
        gat.wait()
        for g in range(CH // 16):
            exg = ex_v[pl.ds(g * 16, 16)]
            for i in range(16):
                ei = g * 16 + i
                exi = exg[i]
                for half in range(OF // 16):
                    sl = pl.ds(half * 16, 16)
                    rows_v[ei, sl] = exi * (rows_v[ei, sl] + me_v[ei, sl])
        pltpu.sync_copy(rows_v, h_sh.at[dst_v], add=True)
        return carry

    lax.fori_loop(0, NCHUNK, _chunk, 0)

    plsc.subcore_barrier()

    @pl.when(sub < NS - 1)
    def _():
        pltpu.sync_copy(h_sh.at[pl.ds(sub * DZ, DZ), :],
                        h_out.at[core, pl.ds(sub * DZ, DZ), :])
        pltpu.sync_copy(d_sh.at[pl.ds(sub * DZ, DZ)],
                        d_out.at[core, pl.ds(sub * DZ, DZ)])

    @pl.when(sub == NS - 1)
    def _():
        pltpu.sync_copy(h_sh.at[pl.ds((NS - 1) * DZ, DTAIL), :],
                        h_out.at[core, pl.ds((NS - 1) * DZ, DTAIL), :])
        pltpu.sync_copy(d_sh.at[pl.ds((NS - 1) * DZ, DTAIL)],
                        d_out.at[core, pl.ds((NS - 1) * DZ, DTAIL)])


def kernel(n_feat, e_feat, edge_index, W_msg_w, W_msg_b, W_w, W_b, a):
    src = edge_index[0]
    dst = edge_index[1]

    asrc, adst, msrc, mdstb = pl.pallas_call(
        _node_kernel,
        out_shape=[
            jax.ShapeDtypeStruct((N, 1), _f32),
            jax.ShapeDtypeStruct((N, 1), _f32),
            jax.ShapeDtypeStruct((N, OF), _f32),
            jax.ShapeDtypeStruct((N, OF), _f32),
        ],
    )(n_feat, W_w, W_msg_w, W_msg_b.reshape(1, OF), a)

    me, ae = pl.pallas_call(
        _edge_kernel,
        grid=(E // BE,),
        in_specs=[
            pl.BlockSpec((BE, EF), lambda i: (i, 0)),
            pl.BlockSpec((OF, DIN), lambda i: (0, 0)),
            pl.BlockSpec((2 * OF, DIN), lambda i: (0, 0)),
            pl.BlockSpec((2 * OF, 1), lambda i: (0, 0)),
            pl.BlockSpec((1, 2 * OF), lambda i: (0, 0)),
        ],
        out_specs=[
            pl.BlockSpec((BE, OF), lambda i: (i, 0)),
            pl.BlockSpec((BE, 1), lambda i: (i, 0)),
        ],
        out_shape=[
            jax.ShapeDtypeStruct((E, OF), _f32),
            jax.ShapeDtypeStruct((E, 1), _f32),
        ],
    )(e_feat, W_msg_w, W_w, a, W_b.reshape(1, 2 * OF))

    hp, dp = _sc_gat(src, dst, asrc.reshape(N), adst.reshape(N),
                     ae.reshape(E), msrc, me)

    out = pl.pallas_call(
        _combine_kernel,
        out_shape=jax.ShapeDtypeStruct((N, OF), _f32),
    )(hp, dp.reshape(NC, N, 1), mdstb)
    return out


# speedup vs baseline: 10.4998x; 1.3165x over previous
import functools

import jax
import jax.numpy as jnp
from jax import lax
from jax.experimental import pallas as pl
from jax.experimental.pallas import tpu as pltpu
from jax.experimental.pallas import tpu_sc as plsc

N = 10000
E = 320000
NF = 128
EF = 16
OF = 32
DIN = 2 * NF + EF
AW = OF + EF

NC = 2
NS = 16
NW = NC * NS
EPT = E // NW
CH = 400
NCHUNK = EPT // CH
DZ = 640
DTAIL = N - (NS - 1) * DZ
DPAD = NS * DZ

BE = 10000

_f32 = jnp.float32


def _node_kernel(n_ref, ww_ref, wm_ref, wmb_ref, a_ref,
                 asrc_ref, adst_ref, msrc_ref, mdstb_ref):
    nf = n_ref[...]
    ww = ww_ref[...]
    wm = wm_ref[...]
    a2 = a_ref[...]
    dn = (((1,), (1,)), ((), ()))
    v_src = lax.dot_general(ww[:, :NF], a2, (((0,), (0,)), ((), ())),
                            preferred_element_type=_f32)
    v_dst = lax.dot_general(ww[:, NF:2 * NF], a2, (((0,), (0,)), ((), ())),
                            preferred_element_type=_f32)
    asrc_ref[...] = lax.dot_general(nf, v_src, (((1,), (0,)), ((), ())),
                                    preferred_element_type=_f32)
    adst_ref[...] = lax.dot_general(nf, v_dst, (((1,), (0,)), ((), ())),
                                    preferred_element_type=_f32)
    msrc_ref[...] = lax.dot_general(nf, wm[:, :NF], dn,
                                    preferred_element_type=_f32)
    mdstb_ref[...] = lax.dot_general(nf, wm[:, NF:2 * NF], dn,
                                     preferred_element_type=_f32) + wmb_ref[...]


def _edge_kernel(e_ref, ww_ref, a_ref, wb_ref, ae_ref):
    ef = e_ref[...]
    v_e = lax.dot_general(ww_ref[...][:, 2 * NF:], a_ref[...],
                          (((0,), (0,)), ((), ())),
                          preferred_element_type=_f32)
    c = lax.dot_general(wb_ref[...], a_ref[...], (((1,), (0,)), ((), ())),
                        preferred_element_type=_f32)
    ae_ref[...] = lax.dot_general(ef, v_e, (((1,), (0,)), ((), ())),
                                  preferred_element_type=_f32) + c[0, 0]


def _combine_kernel(hp_ref, dp_ref, mdstb_ref, wm_ref, out_ref):
    acc = hp_ref[0] + hp_ref[1]
    hs = acc[:, :OF]
    aef = acc[:, OF:]
    me_w = wm_ref[...][:, 2 * NF:]
    h_ef = lax.dot_general(aef, me_w, (((1,), (1,)), ((), ())),
                           preferred_element_type=_f32)
    dn = dp_ref[0] + dp_ref[1]
    valid = dn > 0.0
    safe = jnp.where(valid, dn, 1.0)
    h = jnp.where(valid, (hs + h_ef) / safe + mdstb_ref[...], 0.0)
    out_ref[...] = jnp.maximum(h, 0.0)


_sc_mesh = plsc.VectorSubcoreMesh(core_axis_name="c", subcore_axis_name="s")


@functools.partial(
    pl.kernel,
    out_type=(jax.ShapeDtypeStruct((NC, N, AW), _f32),
              jax.ShapeDtypeStruct((NC, N), _f32)),
    mesh=_sc_mesh,
    scratch_types=[
        pltpu.VMEM_SHARED((N, AW), _f32),
        pltpu.VMEM_SHARED((DPAD,), _f32),
        pltpu.VMEM((N,), _f32),
        pltpu.VMEM((N,), _f32),
        pltpu.VMEM((CH,), jnp.int32),
        pltpu.VMEM((CH,), jnp.int32),
        pltpu.VMEM((CH,), _f32),
        pltpu.VMEM((CH,), _f32),
        pltpu.VMEM((CH, OF), _f32),
        pltpu.VMEM((CH, EF), _f32),
        pltpu.VMEM((CH, AW), _f32),
        pltpu.VMEM((DZ, AW), _f32),
        pltpu.VMEM((DZ,), _f32),
        pltpu.SemaphoreType.DMA,
    ],
    compiler_params=pltpu.CompilerParams(needs_layout_passes=False,
                                         use_tc_tiling_on_sc=False),
)
def _sc_gat(src_hbm, dst_hbm, asrc_hbm, adst_hbm, ae_hbm, msrc_hbm, ef_hbm,
            h_out, d_out,
            h_sh, d_sh, asrc_v, adst_v, src_v, dst_v, ae_v, ex_v,
            rows_v, ef_v, vals_v, zh_v, zd_v, sem):
    core = lax.axis_index("c")
    sub = lax.axis_index("s")
    wid = core * NS + sub

    zvec = jnp.zeros((16,), _f32)

    def _zh(i, carry):
        for part in range(AW // 16):
            zh_v[i, pl.ds(part * 16, 16)] = zvec
        return carry

    lax.fori_loop(0, DZ, _zh, 0)

    def _zd(i, carry):
        zd_v[pl.ds(i * 16, 16)] = zvec
        return carry

    lax.fori_loop(0, DZ // 16, _zd, 0)

    @pl.when(sub < NS - 1)
    def _():
        pltpu.sync_copy(zh_v, h_sh.at[pl.ds(sub * DZ, DZ), :])

    @pl.when(sub == NS - 1)
    def _():
        pltpu.sync_copy(zh_v.at[pl.ds(0, DTAIL), :],
                        h_sh.at[pl.ds((NS - 1) * DZ, DTAIL), :])

    pltpu.sync_copy(zd_v, d_sh.at[pl.ds(sub * DZ, DZ)])

    pltpu.sync_copy(asrc_hbm, asrc_v)
    pltpu.sync_copy(adst_hbm, adst_v)
    plsc.subcore_barrier()

    tile_base = wid * EPT

    def _chunk(chi, carry):
        base = tile_base + chi * CH
        pltpu.sync_copy(src_hbm.at[pl.ds(base, CH)], src_v)
        pltpu.sync_copy(dst_hbm.at[pl.ds(base, CH)], dst_v)
        pltpu.sync_copy(ae_hbm.at[pl.ds(base, CH)], ae_v)
        pltpu.sync_copy(ef_hbm.at[pl.ds(base, CH), :], ef_v)
        gat = pltpu.async_copy(msrc_hbm.at[src_v], rows_v, sem)

        def _logits(g, c2):
            sl = pl.ds(g * 16, 16)
            a_s = plsc.load_gather(asrc_v, [src_v[sl]])
            a_d = plsc.load_gather(adst_v, [dst_v[sl]])
            z = a_s + a_d + ae_v[sl]
            lg = jnp.maximum(z, 0.01 * z)
            ex_v[sl] = jnp.exp(lg)
            return c2

        lax.fori_loop(0, CH // 16, _logits, 0)
        pltpu.sync_copy(ex_v, d_sh.at[dst_v], add=True)
        gat.wait()

        def _scale(g, c2):
            exg = ex_v[pl.ds(g * 16, 16)]
            for i in range(16):
                ei = g * 16 + i
                exi = exg[i]
                for half in range(OF // 16):
                    sl = pl.ds(half * 16, 16)
                    vals_v[ei, sl] = exi * rows_v[ei, sl]
                vals_v[ei, pl.ds(OF, 16)] = exi * ef_v[ei, pl.ds(0, 16)]
            return c2

        lax.fori_loop(0, CH // 16, _scale, 0)
        pltpu.sync_copy(vals_v, h_sh.at[dst_v], add=True)
        return carry

    lax.fori_loop(0, NCHUNK, _chunk, 0)

    plsc.subcore_barrier()

    @pl.when(sub < NS - 1)
    def _():
        pltpu.sync_copy(h_sh.at[pl.ds(sub * DZ, DZ), :],
                        h_out.at[core, pl.ds(sub * DZ, DZ), :])
        pltpu.sync_copy(d_sh.at[pl.ds(sub * DZ, DZ)],
                        d_out.at[core, pl.ds(sub * DZ, DZ)])

    @pl.when(sub == NS - 1)
    def _():
        pltpu.sync_copy(h_sh.at[pl.ds((NS - 1) * DZ, DTAIL), :],
                        h_out.at[core, pl.ds((NS - 1) * DZ, DTAIL), :])
        pltpu.sync_copy(d_sh.at[pl.ds((NS - 1) * DZ, DTAIL)],
                        d_out.at[core, pl.ds((NS - 1) * DZ, DTAIL)])


def kernel(n_feat, e_feat, edge_index, W_msg_w, W_msg_b, W_w, W_b, a):
    src = edge_index[0]
    dst = edge_index[1]

    asrc, adst, msrc, mdstb = pl.pallas_call(
        _node_kernel,
        out_shape=[
            jax.ShapeDtypeStruct((N, 1), _f32),
            jax.ShapeDtypeStruct((N, 1), _f32),
            jax.ShapeDtypeStruct((N, OF), _f32),
            jax.ShapeDtypeStruct((N, OF), _f32),
        ],
    )(n_feat, W_w, W_msg_w, W_msg_b.reshape(1, OF), a)

    ae = pl.pallas_call(
        _edge_kernel,
        grid=(E // BE,),
        in_specs=[
            pl.BlockSpec((BE, EF), lambda i: (i, 0)),
            pl.BlockSpec((2 * OF, DIN), lambda i: (0, 0)),
            pl.BlockSpec((2 * OF, 1), lambda i: (0, 0)),
            pl.BlockSpec((1, 2 * OF), lambda i: (0, 0)),
        ],
        out_specs=pl.BlockSpec((BE, 1), lambda i: (i, 0)),
        out_shape=jax.ShapeDtypeStruct((E, 1), _f32),
    )(e_feat, W_w, a, W_b.reshape(1, 2 * OF))

    hp, dp = _sc_gat(src, dst, asrc.reshape(N), adst.reshape(N),
                     ae.reshape(E), msrc, e_feat)

    out = pl.pallas_call(
        _combine_kernel,
        out_shape=jax.ShapeDtypeStruct((N, OF), _f32),
    )(hp, dp.reshape(NC, N, 1), mdstb, W_msg_w)
    return out


# speedup vs baseline: 11.8538x; 1.1290x over previous
import functools

import jax
import jax.numpy as jnp
from jax import lax
from jax.experimental import pallas as pl
from jax.experimental.pallas import tpu as pltpu
from jax.experimental.pallas import tpu_sc as plsc

N = 10000
E = 320000
NF = 128
EF = 16
OF = 32
DIN = 2 * NF + EF
AW = OF + EF

NC = 2
NS = 16
NW = NC * NS
EPT = E // NW
CH = 400
NCHUNK = EPT // CH
DZ = 640
DTAIL = N - (NS - 1) * DZ
DPAD = NS * DZ

BE = 12800

_f32 = jnp.float32


def _node_kernel(n_ref, ww_ref, wm_ref, wmb_ref, a_ref,
                 asrc_ref, adst_ref, msrc_ref, mdstb_ref):
    nf = n_ref[...]
    ww = ww_ref[...]
    wm = wm_ref[...]
    a2 = a_ref[...]
    dn = (((1,), (1,)), ((), ()))
    v_src = lax.dot_general(ww[:, :NF], a2, (((0,), (0,)), ((), ())),
                            preferred_element_type=_f32)
    v_dst = lax.dot_general(ww[:, NF:2 * NF], a2, (((0,), (0,)), ((), ())),
                            preferred_element_type=_f32)
    asrc_ref[...] = lax.dot_general(nf, v_src, (((1,), (0,)), ((), ())),
                                    preferred_element_type=_f32)
    adst_ref[...] = lax.dot_general(nf, v_dst, (((1,), (0,)), ((), ())),
                                    preferred_element_type=_f32)
    msrc_ref[...] = lax.dot_general(nf, wm[:, :NF], dn,
                                    preferred_element_type=_f32)
    mdstb_ref[...] = lax.dot_general(nf, wm[:, NF:2 * NF], dn,
                                     preferred_element_type=_f32) + wmb_ref[...]


def _edge_kernel(e_ref, ww_ref, a_ref, wb_ref, ae_ref):
    efp = e_ref[...]
    v_e = lax.dot_general(ww_ref[...][:, 2 * NF:], a_ref[...],
                          (((0,), (0,)), ((), ())),
                          preferred_element_type=_f32)
    c = lax.dot_general(wb_ref[...], a_ref[...], (((1,), (0,)), ((), ())),
                        preferred_element_type=_f32)
    vt = jnp.concatenate([v_e] * 8, axis=0)
    row = lax.broadcasted_iota(jnp.int32, (128, 8), 0)
    col = lax.broadcasted_iota(jnp.int32, (128, 8), 1)
    g = jnp.where(row // EF == col, vt, 0.0)
    ae_ref[...] = lax.dot_general(efp, g, (((1,), (0,)), ((), ())),
                                  preferred_element_type=_f32) + c[0, 0]


def _combine_kernel(hp_ref, dp_ref, mdstb_ref, wm_ref, out_ref):
    acc = hp_ref[0] + hp_ref[1]
    hs = acc[:, :OF]
    aef = acc[:, OF:]
    me_w = wm_ref[...][:, 2 * NF:]
    h_ef = lax.dot_general(aef, me_w, (((1,), (1,)), ((), ())),
                           preferred_element_type=_f32)
    dn = dp_ref[0] + dp_ref[1]
    valid = dn > 0.0
    safe = jnp.where(valid, dn, 1.0)
    h = jnp.where(valid, (hs + h_ef) / safe + mdstb_ref[...], 0.0)
    out_ref[...] = jnp.maximum(h, 0.0)


_sc_mesh = plsc.VectorSubcoreMesh(core_axis_name="c", subcore_axis_name="s")


@functools.partial(
    pl.kernel,
    out_type=(jax.ShapeDtypeStruct((NC, N, AW), _f32),
              jax.ShapeDtypeStruct((NC, N), _f32)),
    mesh=_sc_mesh,
    scratch_types=[
        pltpu.VMEM_SHARED((N, AW), _f32),
        pltpu.VMEM_SHARED((DPAD,), _f32),
        pltpu.VMEM((N,), _f32),
        pltpu.VMEM((N,), _f32),
        pltpu.VMEM((CH,), jnp.int32),
        pltpu.VMEM((CH,), jnp.int32),
        pltpu.VMEM((CH,), _f32),
        pltpu.VMEM((CH,), _f32),
        pltpu.VMEM((CH, OF), _f32),
        pltpu.VMEM((CH // 8, 128), _f32),
        pltpu.VMEM((CH, AW), _f32),
        pltpu.VMEM((DZ, AW), _f32),
        pltpu.VMEM((DZ,), _f32),
        pltpu.SemaphoreType.DMA,
    ],
    compiler_params=pltpu.CompilerParams(needs_layout_passes=False,
                                         use_tc_tiling_on_sc=False),
)
def _sc_gat(src_hbm, dst_hbm, asrc_hbm, adst_hbm, ae_hbm, msrc_hbm, ef_hbm,
            h_out, d_out,
            h_sh, d_sh, asrc_v, adst_v, src_v, dst_v, ae_v, ex_v,
            rows_v, ef_v, vals_v, zh_v, zd_v, sem):
    core = lax.axis_index("c")
    sub = lax.axis_index("s")
    wid = core * NS + sub

    zvec = jnp.zeros((16,), _f32)

    def _zh(i, carry):
        for part in range(AW // 16):
            zh_v[i, pl.ds(part * 16, 16)] = zvec
        return carry

    lax.fori_loop(0, DZ, _zh, 0)

    def _zd(i, carry):
        zd_v[pl.ds(i * 16, 16)] = zvec
        return carry

    lax.fori_loop(0, DZ // 16, _zd, 0)

    @pl.when(sub < NS - 1)
    def _():
        pltpu.sync_copy(zh_v, h_sh.at[pl.ds(sub * DZ, DZ), :])

    @pl.when(sub == NS - 1)
    def _():
        pltpu.sync_copy(zh_v.at[pl.ds(0, DTAIL), :],
                        h_sh.at[pl.ds((NS - 1) * DZ, DTAIL), :])

    pltpu.sync_copy(zd_v, d_sh.at[pl.ds(sub * DZ, DZ)])

    pltpu.sync_copy(asrc_hbm, asrc_v)
    pltpu.sync_copy(adst_hbm, adst_v)
    plsc.subcore_barrier()

    tile_base = wid * EPT

    def _chunk(chi, carry):
        base = tile_base + chi * CH
        pltpu.sync_copy(src_hbm.at[pl.ds(base, CH)], src_v)
        pltpu.sync_copy(dst_hbm.at[pl.ds(base, CH)], dst_v)
        pltpu.sync_copy(ae_hbm.at[pl.ds(base, CH)], ae_v)
        pltpu.sync_copy(ef_hbm.at[pl.ds(base // 8, CH // 8), :], ef_v)
        gat = pltpu.async_copy(msrc_hbm.at[src_v], rows_v, sem)

        def _logits(g, c2):
            sl = pl.ds(g * 16, 16)
            a_s = plsc.load_gather(asrc_v, [src_v[sl]])
            a_d = plsc.load_gather(adst_v, [dst_v[sl]])
            z = a_s + a_d + ae_v[sl]
            lg = jnp.maximum(z, 0.01 * z)
            ex_v[sl] = jnp.exp(lg)
            return c2

        lax.fori_loop(0, CH // 16, _logits, 0)
        pltpu.sync_copy(ex_v, d_sh.at[dst_v], add=True)
        gat.wait()

        def _scale(g, c2):
            exg = ex_v[pl.ds(g * 16, 16)]
            for i in range(16):
                ei = g * 16 + i
                exi = exg[i]
                for half in range(OF // 16):
                    sl = pl.ds(half * 16, 16)
                    vals_v[ei, sl] = exi * rows_v[ei, sl]
                vals_v[ei, pl.ds(OF, 16)] = exi * ef_v[2 * g + i // 8,
                                                       pl.ds((i % 8) * EF, EF)]
            return c2

        lax.fori_loop(0, CH // 16, _scale, 0)
        pltpu.sync_copy(vals_v, h_sh.at[dst_v], add=True)
        return carry

    lax.fori_loop(0, NCHUNK, _chunk, 0)

    plsc.subcore_barrier()

    @pl.when(sub < NS - 1)
    def _():
        pltpu.sync_copy(h_sh.at[pl.ds(sub * DZ, DZ), :],
                        h_out.at[core, pl.ds(sub * DZ, DZ), :])
        pltpu.sync_copy(d_sh.at[pl.ds(sub * DZ, DZ)],
                        d_out.at[core, pl.ds(sub * DZ, DZ)])

    @pl.when(sub == NS - 1)
    def _():
        pltpu.sync_copy(h_sh.at[pl.ds((NS - 1) * DZ, DTAIL), :],
                        h_out.at[core, pl.ds((NS - 1) * DZ, DTAIL), :])
        pltpu.sync_copy(d_sh.at[pl.ds((NS - 1) * DZ, DTAIL)],
                        d_out.at[core, pl.ds((NS - 1) * DZ, DTAIL)])


def kernel(n_feat, e_feat, edge_index, W_msg_w, W_msg_b, W_w, W_b, a):
    src = edge_index[0]
    dst = edge_index[1]

    asrc, adst, msrc, mdstb = pl.pallas_call(
        _node_kernel,
        out_shape=[
            jax.ShapeDtypeStruct((N, 1), _f32),
            jax.ShapeDtypeStruct((N, 1), _f32),
            jax.ShapeDtypeStruct((N, OF), _f32),
            jax.ShapeDtypeStruct((N, OF), _f32),
        ],
    )(n_feat, W_w, W_msg_w, W_msg_b.reshape(1, OF), a)

    ef_lin = e_feat.reshape(E // 8, 128)

    ae_p = pl.pallas_call(
        _edge_kernel,
        grid=(E // BE,),
        in_specs=[
            pl.BlockSpec((BE // 8, 128), lambda i: (i, 0)),
            pl.BlockSpec((2 * OF, DIN), lambda i: (0, 0)),
            pl.BlockSpec((2 * OF, 1), lambda i: (0, 0)),
            pl.BlockSpec((1, 2 * OF), lambda i: (0, 0)),
        ],
        out_specs=pl.BlockSpec((BE // 8, 8), lambda i: (i, 0)),
        out_shape=jax.ShapeDtypeStruct((E // 8, 8), _f32),
    )(ef_lin, W_w, a, W_b.reshape(1, 2 * OF))

    hp, dp = _sc_gat(src, dst, asrc.reshape(N), adst.reshape(N),
                     ae_p.reshape(E), msrc, ef_lin)

    out = pl.pallas_call(
        _combine_kernel,
        out_shape=jax.ShapeDtypeStruct((N, OF), _f32),
    )(hp, dp.reshape(NC, N, 1), mdstb, W_msg_w)
    return out
